# baseline (speedup 1.0000x reference)
import functools

import jax
import jax.numpy as jnp
from jax import lax
from jax.experimental import pallas as pl
from jax.experimental.pallas import tpu as pltpu
from jax.experimental.pallas import tpu_sc as plsc

N = 10000
E = 320000
R = 8
D = 128

NC, NS, L = 2, 16, 16
NW = NC * NS
EB = 128
E_PAD = 327680
ROWS = E_PAD // EB
RPW = ROWS // NW

AR = 10240
ARS = AR // NS
CH = 16
RC0 = 112
RC1 = 48
NB = 80128
NBS = NB // NS
JUNK_B = N * R

BN = 2000
NT = N // BN

_mesh = plsc.VectorSubcoreMesh(core_axis_name="c", subcore_axis_name="s")
_sc_params = pltpu.CompilerParams(use_tc_tiling_on_sc=False)


def _mm_body(x_ref, w_ref, o_ref):
    o_ref[0] = jnp.dot(x_ref[...], w_ref[0],
                       preferred_element_type=jnp.float32,
                       precision=lax.Precision.HIGHEST)


def _transform(x, w_all):
    return pl.pallas_call(
        _mm_body,
        grid=(NT, R + 1),
        in_specs=[pl.BlockSpec((BN, D), lambda n, r: (n, 0)),
                  pl.BlockSpec((1, D, D), lambda n, r: (r, 0, 0))],
        out_specs=pl.BlockSpec((1, BN, D), lambda n, r: (r, n, 0)),
        out_shape=jax.ShapeDtypeStruct((R + 1, N, D), jnp.float32),
    )(x, w_all)


def _inv_body(c_ref, o_ref):
    o_ref[...] = 1.0 / jnp.maximum(c_ref[0] + c_ref[1], 1.0)


def _invert_counts(cnt_parts):
    c2 = cnt_parts.reshape(NC, NB * 16 // D, D)
    m = c2.shape[1]
    inv = pl.pallas_call(
        _inv_body,
        grid=(4,),
        in_specs=[pl.BlockSpec((NC, m // 4, D), lambda i: (0, i, 0))],
        out_specs=pl.BlockSpec((m // 4, D), lambda i: (i, 0)),
        out_shape=jax.ShapeDtypeStruct((m, D), jnp.float32),
    )(c2)
    return inv.reshape(NB, 16)


def _combine_body(root_ref, p_ref, b_ref, o_ref, *, relu):
    v = root_ref[0] + p_ref[0] + p_ref[1] + b_ref[...]
    o_ref[...] = jnp.maximum(v, 0.0) if relu else v


def _combine(h_all, parts, b, relu):
    return pl.pallas_call(
        functools.partial(_combine_body, relu=relu),
        grid=(NT,),
        in_specs=[pl.BlockSpec((1, BN, D), lambda n: (R, n, 0)),
                  pl.BlockSpec((NC, BN, D), lambda n: (0, n, 0)),
                  pl.BlockSpec((1, D), lambda n: (0, 0))],
        out_specs=pl.BlockSpec((BN, D), lambda n: (n, 0)),
        out_shape=jax.ShapeDtypeStruct((N, D), jnp.float32),
    )(h_all, parts, b.reshape(1, D))


@functools.partial(
    pl.kernel, mesh=_mesh,
    out_type=jax.ShapeDtypeStruct((NC, NB, 16), jnp.float32),
    scratch_types=[
        pltpu.VMEM((RPW, EB), jnp.int32),
        pltpu.VMEM((EB, 16), jnp.float32),
        pltpu.VMEM_SHARED((NB, 16), jnp.float32),
    ],
    compiler_params=_sc_params,
)
def _sc_counts(key_hbm, zero_hbm, out_hbm, key_v, ones_v, cnt_sh):
    c = lax.axis_index("c")
    s = lax.axis_index("s")
    w = c * NS + s
    pltpu.sync_copy(zero_hbm.at[pl.ds(s * NBS, NBS)],
                    cnt_sh.at[pl.ds(s * NBS, NBS)])
    pltpu.sync_copy(key_hbm.at[pl.ds(w * RPW, RPW)], key_v)

    @pl.loop(0, EB)
    def _fill(e):
        ones_v[e, :] = jnp.full((16,), 1.0, jnp.float32)

    plsc.subcore_barrier()

    @pl.loop(0, RPW)
    def _scatter(j):
        pltpu.sync_copy(ones_v, cnt_sh.at[key_v.at[j]], add=True)

    plsc.subcore_barrier()
    pltpu.sync_copy(cnt_sh.at[pl.ds(s * NBS, NBS)],
                    out_hbm.at[c].at[pl.ds(s * NBS, NBS)])


def _make_agg(mat_w):
    out_types = [jax.ShapeDtypeStruct((NC, AR, D), jnp.float32)]
    if mat_w:
        out_types.append(jax.ShapeDtypeStruct((ROWS, EB, 16), jnp.float32))
    scratch = [
        pltpu.VMEM((CH, EB), jnp.int32),
        pltpu.VMEM((CH, EB), jnp.int32),
        pltpu.VMEM((CH, EB), jnp.int32),
        pltpu.VMEM((2, EB, D), jnp.float32),
        pltpu.VMEM((2, EB, 16), jnp.float32),
        pltpu.VMEM_SHARED((AR, D), jnp.float32),
        [pltpu.SemaphoreType.DMA] * 2,
        [pltpu.SemaphoreType.DMA] * 2,
    ]
    if mat_w:
        scratch.append([pltpu.SemaphoreType.DMA] * 2)

    @functools.partial(
        pl.kernel, mesh=_mesh,
        out_type=tuple(out_types) if mat_w else out_types[0],
        scratch_types=scratch,
        compiler_params=_sc_params,
    )
    def _agg(hflat_hbm, w_hbm, g_hbm, key_hbm, dst_hbm, zero_hbm, *rest):
        if mat_w:
            (out_hbm, wout_hbm, g_v, key_v, dst_v, rows_v, inv_v, acc_sh,
             gsems, ssems, wsems) = rest
        else:
            (out_hbm, g_v, key_v, dst_v, rows_v, inv_v, acc_sh,
             gsems, ssems) = rest
        c = lax.axis_index("c")
        s = lax.axis_index("s")
        with jax.named_scope("agg_init"):
            pltpu.sync_copy(zero_hbm.at[pl.ds(s * ARS, ARS)],
                            acc_sh.at[pl.ds(s * ARS, ARS)])
            plsc.subcore_barrier()

        def _issue_gathers(j, glob, buf):
            pltpu.async_copy(hflat_hbm.at[g_v.at[j]], rows_v.at[buf],
                             gsems[buf])
            if mat_w:
                pltpu.async_copy(w_hbm.at[key_v.at[j]], inv_v.at[buf],
                                 gsems[buf])
            else:
                pltpu.async_copy(w_hbm.at[glob], inv_v.at[buf], gsems[buf])

        def _wait_gathers(buf):
            pltpu.make_async_copy(hflat_hbm.at[g_v.at[0]], rows_v.at[buf],
                                  gsems[buf]).wait()
            if mat_w:
                pltpu.make_async_copy(w_hbm.at[key_v.at[0]], inv_v.at[buf],
                                      gsems[buf]).wait()
            else:
                pltpu.make_async_copy(w_hbm.at[0], inv_v.at[buf],
                                      gsems[buf]).wait()

        def _scale(buf):
            @pl.loop(0, EB)
            def _(e):
                iv = inv_v[buf, e, :]
                for t in range(D // 16):
                    sl = pl.ds(t * 16, 16)
                    rows_v[buf, e, sl] = rows_v[buf, e, sl] * iv

        def _write_w(glob, buf):
            if mat_w:
                pltpu.async_copy(inv_v.at[buf], wout_hbm.at[glob], wsems[buf])

        def _wait_w(buf):
            if mat_w:
                pltpu.make_async_copy(inv_v.at[buf], wout_hbm.at[0],
                                      wsems[buf]).wait()

        def _scatter(j, buf):
            pltpu.async_copy(rows_v.at[buf], acc_sh.at[dst_v.at[j]],
                             ssems[buf], add=True)

        def _wait_scatter(buf):
            pltpu.make_async_copy(rows_v.at[buf], acc_sh.at[dst_v.at[0]],
                                  ssems[buf]).wait()

        def _run_core(n_tile_rows, core_base):
            @pl.loop(0, n_tile_rows // CH)
            def _group(grp):
                base = core_base + s * n_tile_rows + grp * CH
                pltpu.sync_copy(g_hbm.at[pl.ds(base, CH)], g_v)
                pltpu.sync_copy(key_hbm.at[pl.ds(base, CH)], key_v)
                pltpu.sync_copy(dst_hbm.at[pl.ds(base, CH)], dst_v)

                _issue_gathers(0, base + 0, 0)
                _issue_gathers(1, base + 1, 1)

                @pl.loop(2, CH, step=2)
                def _block(j):
                    _wait_gathers(0)
                    _scale(0)
                    _write_w(base + j - 2, 0)
                    _scatter(j - 2, 0)
                    _wait_gathers(1)
                    _scale(1)
                    _write_w(base + j - 1, 1)
                    _scatter(j - 1, 1)
                    _wait_scatter(0)
                    _wait_w(0)
                    _issue_gathers(j, base + j, 0)
                    _wait_scatter(1)
                    _wait_w(1)
                    _issue_gathers(j + 1, base + j + 1, 1)

                _wait_gathers(0)
                _scale(0)
                _write_w(base + CH - 2, 0)
                _scatter(CH - 2, 0)
                _wait_gathers(1)
                _scale(1)
                _write_w(base + CH - 1, 1)
                _scatter(CH - 1, 1)
                _wait_scatter(0)
                _wait_scatter(1)
                _wait_w(0)
                _wait_w(1)

        with jax.named_scope("agg_edges"):
            @pl.when(c == 0)
            def _():
                _run_core(RC0, 0)

            @pl.when(c == 1)
            def _():
                _run_core(RC1, NS * RC0)

        with jax.named_scope("agg_out"):
            plsc.subcore_barrier()
            pltpu.sync_copy(acc_sh.at[pl.ds(s * ARS, ARS)],
                            out_hbm.at[c].at[pl.ds(s * ARS, ARS)])

    return _agg


_sc_agg_w = _make_agg(True)
_sc_agg_r = _make_agg(False)


def kernel(x, edge_index, edge_type, W_rel1, W_root1, b1, W_rel2, W_root2, b2):
    src = edge_index[0]
    dst = edge_index[1]
    pad = E_PAD - E

    g = edge_type * N + src
    keyb = dst * R + edge_type
    pad_i = jnp.arange(pad, dtype=jnp.int32)
    g_p = jnp.concatenate(
        [g, jnp.zeros((pad,), jnp.int32)]).reshape(ROWS, EB)
    key_p = jnp.concatenate(
        [keyb, JUNK_B + pad_i % (NB - JUNK_B)]).reshape(ROWS, EB)
    dst_p = jnp.concatenate(
        [dst, N + pad_i % (AR - N)]).reshape(ROWS, EB)

    z_cnt = jnp.zeros((NB, 16), jnp.float32)
    z_acc = jnp.zeros((AR, D), jnp.float32)

    cnt_parts = _sc_counts(key_p, z_cnt)
    inv16 = _invert_counts(cnt_parts)

    w_all1 = jnp.concatenate([W_rel1, W_root1[None]], axis=0)
    w_all2 = jnp.concatenate([W_rel2, W_root2[None]], axis=0)

    h_all1 = _transform(x, w_all1)
    parts1, w16 = _sc_agg_w(h_all1.reshape((R + 1) * N, D), inv16,
                            g_p, key_p, dst_p, z_acc)
    h1 = _combine(h_all1, parts1, b1, relu=True)

    h_all2 = _transform(h1, w_all2)
    parts2 = _sc_agg_r(h_all2.reshape((R + 1) * N, D), w16,
                       g_p, key_p, dst_p, z_acc)
    return _combine(h_all2, parts2, b2, relu=False)


# speedup vs baseline: 1.2002x; 1.2002x over previous
import functools

import jax
import jax.numpy as jnp
from jax import lax
from jax.experimental import pallas as pl
from jax.experimental.pallas import tpu as pltpu
from jax.experimental.pallas import tpu_sc as plsc

N = 10000
E = 320000
R = 8
D = 128

NC, NS, L = 2, 16, 16
NW = NC * NS
EB = 128
E_PAD = 327680
ROWS = E_PAD // EB
RPW = ROWS // NW

AR = 10240
ARS = AR // NS
CH = 16
RC0 = 96
RC1 = 64
NB = 80128
NBS = NB // NS
JUNK_B = N * R

BN = 2000
NT = N // BN

_mesh = plsc.VectorSubcoreMesh(core_axis_name="c", subcore_axis_name="s")
_sc_params = pltpu.CompilerParams(use_tc_tiling_on_sc=False)


def _mm_body(x_ref, w_ref, o_ref):
    o_ref[0] = jnp.dot(x_ref[...], w_ref[0],
                       preferred_element_type=jnp.float32,
                       precision=lax.Precision.HIGHEST)


def _transform(x, w_all):
    return pl.pallas_call(
        _mm_body,
        grid=(NT, R + 1),
        in_specs=[pl.BlockSpec((BN, D), lambda n, r: (n, 0)),
                  pl.BlockSpec((1, D, D), lambda n, r: (r, 0, 0))],
        out_specs=pl.BlockSpec((1, BN, D), lambda n, r: (r, n, 0)),
        out_shape=jax.ShapeDtypeStruct((R + 1, N, D), jnp.float32),
    )(x, w_all)


def _inv_body(c_ref, o_ref):
    o_ref[...] = 1.0 / jnp.maximum(c_ref[0] + c_ref[1], 1.0)


def _invert_counts(cnt_parts):
    c2 = cnt_parts.reshape(NC, NB * 16 // D, D)
    m = c2.shape[1]
    inv = pl.pallas_call(
        _inv_body,
        grid=(4,),
        in_specs=[pl.BlockSpec((NC, m // 4, D), lambda i: (0, i, 0))],
        out_specs=pl.BlockSpec((m // 4, D), lambda i: (i, 0)),
        out_shape=jax.ShapeDtypeStruct((m, D), jnp.float32),
    )(c2)
    return inv.reshape(NB, 16)


def _combine_body(root_ref, p_ref, b_ref, o_ref, *, relu):
    v = root_ref[0] + p_ref[0] + p_ref[1] + b_ref[...]
    o_ref[...] = jnp.maximum(v, 0.0) if relu else v


def _combine(h_all, parts, b, relu):
    return pl.pallas_call(
        functools.partial(_combine_body, relu=relu),
        grid=(NT,),
        in_specs=[pl.BlockSpec((1, BN, D), lambda n: (R, n, 0)),
                  pl.BlockSpec((NC, BN, D), lambda n: (0, n, 0)),
                  pl.BlockSpec((1, D), lambda n: (0, 0))],
        out_specs=pl.BlockSpec((BN, D), lambda n: (n, 0)),
        out_shape=jax.ShapeDtypeStruct((N, D), jnp.float32),
    )(h_all, parts, b.reshape(1, D))


@functools.partial(
    pl.kernel, mesh=_mesh,
    out_type=jax.ShapeDtypeStruct((NC, NB, 16), jnp.float32),
    scratch_types=[
        pltpu.VMEM((RPW, EB), jnp.int32),
        pltpu.VMEM((EB, 16), jnp.float32),
        pltpu.VMEM_SHARED((NB, 16), jnp.float32),
    ],
    compiler_params=_sc_params,
)
def _sc_counts(key_hbm, zero_hbm, out_hbm, key_v, ones_v, cnt_sh):
    c = lax.axis_index("c")
    s = lax.axis_index("s")
    w = c * NS + s
    pltpu.sync_copy(zero_hbm.at[pl.ds(s * NBS, NBS)],
                    cnt_sh.at[pl.ds(s * NBS, NBS)])
    pltpu.sync_copy(key_hbm.at[pl.ds(w * RPW, RPW)], key_v)

    @pl.loop(0, EB)
    def _fill(e):
        ones_v[e, :] = jnp.full((16,), 1.0, jnp.float32)

    plsc.subcore_barrier()

    @pl.loop(0, RPW)
    def _scatter(j):
        pltpu.sync_copy(ones_v, cnt_sh.at[key_v.at[j]], add=True)

    plsc.subcore_barrier()
    pltpu.sync_copy(cnt_sh.at[pl.ds(s * NBS, NBS)],
                    out_hbm.at[c].at[pl.ds(s * NBS, NBS)])


@functools.partial(
    pl.kernel, mesh=_mesh,
    out_type=jax.ShapeDtypeStruct((NC, AR, D), jnp.float32),
    scratch_types=[
        pltpu.VMEM((CH, EB), jnp.int32),
        pltpu.VMEM((CH, EB), jnp.int32),
        pltpu.VMEM((CH, EB), jnp.int32),
        pltpu.VMEM((2, EB, D), jnp.float32),
        pltpu.VMEM((2, EB, 16), jnp.float32),
        pltpu.VMEM_SHARED((AR, D), jnp.float32),
        [pltpu.SemaphoreType.DMA] * 2,
        [pltpu.SemaphoreType.DMA] * 2,
    ],
    compiler_params=_sc_params,
)
def _sc_agg(hflat_hbm, inv_hbm, g_hbm, key_hbm, dst_hbm, zero_hbm, out_hbm,
            g_v, key_v, dst_v, rows_v, inv_v, acc_sh, gsems, ssems):
    c = lax.axis_index("c")
    s = lax.axis_index("s")
    with jax.named_scope("agg_init"):
        pltpu.sync_copy(zero_hbm.at[pl.ds(s * ARS, ARS)],
                        acc_sh.at[pl.ds(s * ARS, ARS)])
        plsc.subcore_barrier()

    def _issue_gathers(j, buf):
        pltpu.async_copy(hflat_hbm.at[g_v.at[j]], rows_v.at[buf], gsems[buf])
        pltpu.async_copy(inv_hbm.at[key_v.at[j]], inv_v.at[buf], gsems[buf])

    def _wait_gathers(buf):
        pltpu.make_async_copy(hflat_hbm.at[g_v.at[0]], rows_v.at[buf],
                              gsems[buf]).wait()
        pltpu.make_async_copy(inv_hbm.at[key_v.at[0]], inv_v.at[buf],
                              gsems[buf]).wait()

    def _scale(buf):
        @pl.loop(0, EB)
        def _(e):
            iv = inv_v[buf, e, :]
            for t in range(D // 16):
                sl = pl.ds(t * 16, 16)
                rows_v[buf, e, sl] = rows_v[buf, e, sl] * iv

    def _scatter(j, buf):
        pltpu.async_copy(rows_v.at[buf], acc_sh.at[dst_v.at[j]],
                         ssems[buf], add=True)

    def _wait_scatter(buf):
        pltpu.make_async_copy(rows_v.at[buf], acc_sh.at[dst_v.at[0]],
                              ssems[buf]).wait()

    def _run_core(n_tile_rows, core_base):
        @pl.loop(0, n_tile_rows // CH)
        def _group(grp):
            base = core_base + s * n_tile_rows + grp * CH
            pltpu.sync_copy(g_hbm.at[pl.ds(base, CH)], g_v)
            pltpu.sync_copy(key_hbm.at[pl.ds(base, CH)], key_v)
            pltpu.sync_copy(dst_hbm.at[pl.ds(base, CH)], dst_v)

            _issue_gathers(0, 0)
            _issue_gathers(1, 1)

            @pl.loop(2, CH, step=2)
            def _block(j):
                _wait_gathers(0)
                _scale(0)
                _scatter(j - 2, 0)
                _wait_gathers(1)
                _scale(1)
                _scatter(j - 1, 1)
                _wait_scatter(0)
                _issue_gathers(j, 0)
                _wait_scatter(1)
                _issue_gathers(j + 1, 1)

            _wait_gathers(0)
            _scale(0)
            _scatter(CH - 2, 0)
            _wait_gathers(1)
            _scale(1)
            _scatter(CH - 1, 1)
            _wait_scatter(0)
            _wait_scatter(1)

    with jax.named_scope("agg_edges"):
        @pl.when(c == 0)
        def _():
            _run_core(RC0, 0)

        @pl.when(c == 1)
        def _():
            _run_core(RC1, NS * RC0)

    with jax.named_scope("agg_out"):
        plsc.subcore_barrier()

        @pl.loop(0, ARS // EB)
        def _out(k):
            row = s * ARS + k * EB
            pltpu.sync_copy(acc_sh.at[pl.ds(row, EB)], rows_v.at[0])
            pltpu.sync_copy(rows_v.at[0], out_hbm.at[c].at[pl.ds(row, EB)])


def kernel(x, edge_index, edge_type, W_rel1, W_root1, b1, W_rel2, W_root2, b2):
    src = edge_index[0]
    dst = edge_index[1]
    pad = E_PAD - E

    g = edge_type * N + src
    keyb = dst * R + edge_type
    pad_i = jnp.arange(pad, dtype=jnp.int32)
    g_p = jnp.concatenate(
        [g, jnp.zeros((pad,), jnp.int32)]).reshape(ROWS, EB)
    key_p = jnp.concatenate(
        [keyb, JUNK_B + pad_i % (NB - JUNK_B)]).reshape(ROWS, EB)
    dst_p = jnp.concatenate(
        [dst, N + pad_i % (AR - N)]).reshape(ROWS, EB)

    z_cnt = jnp.zeros((NB, 16), jnp.float32)
    z_acc = jnp.zeros((AR, D), jnp.float32)

    cnt_parts = _sc_counts(key_p, z_cnt)
    inv16 = _invert_counts(cnt_parts)

    w_all1 = jnp.concatenate([W_rel1, W_root1[None]], axis=0)
    w_all2 = jnp.concatenate([W_rel2, W_root2[None]], axis=0)

    h_all1 = _transform(x, w_all1)
    parts1 = _sc_agg(h_all1.reshape((R + 1) * N, D), inv16,
                     g_p, key_p, dst_p, z_acc)
    h1 = _combine(h_all1, parts1, b1, relu=True)

    h_all2 = _transform(h1, w_all2)
    parts2 = _sc_agg(h_all2.reshape((R + 1) * N, D), inv16,
                     g_p, key_p, dst_p, z_acc)
    return _combine(h_all2, parts2, b2, relu=False)


# speedup vs baseline: 1.3385x; 1.1152x over previous
import functools

import jax
import jax.numpy as jnp
from jax import lax
from jax.experimental import pallas as pl
from jax.experimental.pallas import tpu as pltpu
from jax.experimental.pallas import tpu_sc as plsc

N = 10000
E = 320000
R = 8
D = 128

NC, NS, L = 2, 16, 16
NW = NC * NS
EB = 128
E_PAD = 327680
ROWS = E_PAD // EB
RPW = ROWS // NW

AR = 10240
ARS = AR // NS
CH = 16
RC0 = 128
RC1 = 32
NB = 80128
NBS = NB // NS
JUNK_B = N * R

BN = 2000
NT = N // BN

_mesh = plsc.VectorSubcoreMesh(core_axis_name="c", subcore_axis_name="s")
_sc_params = pltpu.CompilerParams(use_tc_tiling_on_sc=False)


def _mm_body(x_ref, w_ref, o_ref):
    o_ref[0] = jnp.dot(x_ref[...], w_ref[0],
                       preferred_element_type=jnp.float32,
                       precision=lax.Precision.HIGHEST)


def _transform(x, w_all):
    return pl.pallas_call(
        _mm_body,
        grid=(NT, R + 1),
        in_specs=[pl.BlockSpec((BN, D), lambda n, r: (n, 0)),
                  pl.BlockSpec((1, D, D), lambda n, r: (r, 0, 0))],
        out_specs=pl.BlockSpec((1, BN, D), lambda n, r: (r, n, 0)),
        out_shape=jax.ShapeDtypeStruct((R + 1, N, D), jnp.float32),
    )(x, w_all)


def _inv_body(c_ref, o_ref):
    o_ref[...] = 1.0 / jnp.maximum(c_ref[0] + c_ref[1], 1.0)


def _invert_counts(cnt_parts):
    c2 = cnt_parts.reshape(NC, NB * 16 // D, D)
    m = c2.shape[1]
    inv = pl.pallas_call(
        _inv_body,
        grid=(4,),
        in_specs=[pl.BlockSpec((NC, m // 4, D), lambda i: (0, i, 0))],
        out_specs=pl.BlockSpec((m // 4, D), lambda i: (i, 0)),
        out_shape=jax.ShapeDtypeStruct((m, D), jnp.float32),
    )(c2)
    return inv.reshape(NB, 16)


def _combine_body(root_ref, p_ref, b_ref, o_ref, *, relu):
    v = root_ref[0] + p_ref[0] + p_ref[1] + b_ref[...]
    o_ref[...] = jnp.maximum(v, 0.0) if relu else v


def _combine(h_all, parts, b, relu):
    return pl.pallas_call(
        functools.partial(_combine_body, relu=relu),
        grid=(NT,),
        in_specs=[pl.BlockSpec((1, BN, D), lambda n: (R, n, 0)),
                  pl.BlockSpec((NC, BN, D), lambda n: (0, n, 0)),
                  pl.BlockSpec((1, D), lambda n: (0, 0))],
        out_specs=pl.BlockSpec((BN, D), lambda n: (n, 0)),
        out_shape=jax.ShapeDtypeStruct((N, D), jnp.float32),
    )(h_all, parts, b.reshape(1, D))


@functools.partial(
    pl.kernel, mesh=_mesh,
    out_type=jax.ShapeDtypeStruct((NC, NB, 16), jnp.float32),
    scratch_types=[
        pltpu.VMEM((RPW, EB), jnp.int32),
        pltpu.VMEM((EB, 16), jnp.float32),
        pltpu.VMEM_SHARED((NB, 16), jnp.float32),
    ],
    compiler_params=_sc_params,
)
def _sc_counts(key_hbm, zero_hbm, out_hbm, key_v, ones_v, cnt_sh):
    c = lax.axis_index("c")
    s = lax.axis_index("s")
    w = c * NS + s
    pltpu.sync_copy(zero_hbm.at[pl.ds(s * NBS, NBS)],
                    cnt_sh.at[pl.ds(s * NBS, NBS)])
    pltpu.sync_copy(key_hbm.at[pl.ds(w * RPW, RPW)], key_v)

    @pl.loop(0, EB)
    def _fill(e):
        ones_v[e, :] = jnp.full((16,), 1.0, jnp.float32)

    plsc.subcore_barrier()

    @pl.loop(0, RPW)
    def _scatter(j):
        pltpu.sync_copy(ones_v, cnt_sh.at[key_v.at[j]], add=True)

    plsc.subcore_barrier()
    pltpu.sync_copy(cnt_sh.at[pl.ds(s * NBS, NBS)],
                    out_hbm.at[c].at[pl.ds(s * NBS, NBS)])


@functools.partial(
    pl.kernel, mesh=_mesh,
    out_type=jax.ShapeDtypeStruct((NC, AR, D), jnp.float32),
    scratch_types=[
        pltpu.VMEM((CH, EB), jnp.int32),
        pltpu.VMEM((CH, EB), jnp.int32),
        pltpu.VMEM((CH, EB), jnp.int32),
        pltpu.VMEM((2, EB, D), jnp.float32),
        pltpu.VMEM((2, EB, 16), jnp.float32),
        pltpu.VMEM_SHARED((AR, D), jnp.float32),
        [pltpu.SemaphoreType.DMA] * 2,
        [pltpu.SemaphoreType.DMA] * 2,
    ],
    compiler_params=_sc_params,
)
def _sc_agg(hflat_hbm, inv_hbm, g_hbm, key_hbm, dst_hbm, zero_hbm, out_hbm,
            g_v, key_v, dst_v, rows_v, inv_v, acc_sh, gsems, ssems):
    c = lax.axis_index("c")
    s = lax.axis_index("s")
    with jax.named_scope("agg_init"):
        pltpu.sync_copy(zero_hbm.at[pl.ds(s * ARS, ARS)],
                        acc_sh.at[pl.ds(s * ARS, ARS)])
        plsc.subcore_barrier()

    def _issue_gathers(j, buf):
        pltpu.async_copy(hflat_hbm.at[g_v.at[j]], rows_v.at[buf], gsems[buf])
        pltpu.async_copy(inv_hbm.at[key_v.at[j]], inv_v.at[buf], gsems[buf])

    def _wait_gathers(buf):
        pltpu.make_async_copy(hflat_hbm.at[g_v.at[0]], rows_v.at[buf],
                              gsems[buf]).wait()
        pltpu.make_async_copy(inv_hbm.at[key_v.at[0]], inv_v.at[buf],
                              gsems[buf]).wait()

    def _scale(buf):
        @pl.loop(0, EB)
        def _(e):
            iv = inv_v[buf, e, :]
            for t in range(D // 16):
                sl = pl.ds(t * 16, 16)
                rows_v[buf, e, sl] = rows_v[buf, e, sl] * iv

    def _scatter(j, buf):
        pltpu.async_copy(rows_v.at[buf], acc_sh.at[dst_v.at[j]],
                         ssems[buf], add=True)

    def _wait_scatter(buf):
        pltpu.make_async_copy(rows_v.at[buf], acc_sh.at[dst_v.at[0]],
                              ssems[buf]).wait()

    def _run_core(n_tile_rows, core_base):
        @pl.loop(0, n_tile_rows // CH)
        def _group(grp):
            base = core_base + s * n_tile_rows + grp * CH
            pltpu.sync_copy(g_hbm.at[pl.ds(base, CH)], g_v)
            pltpu.sync_copy(key_hbm.at[pl.ds(base, CH)], key_v)
            pltpu.sync_copy(dst_hbm.at[pl.ds(base, CH)], dst_v)

            _issue_gathers(0, 0)
            _issue_gathers(1, 1)

            @pl.loop(2, CH, step=2)
            def _block(j):
                _wait_gathers(0)
                _scale(0)
                _scatter(j - 2, 0)
                _wait_gathers(1)
                _scale(1)
                _scatter(j - 1, 1)
                _wait_scatter(0)
                _issue_gathers(j, 0)
                _wait_scatter(1)
                _issue_gathers(j + 1, 1)

            _wait_gathers(0)
            _scale(0)
            _scatter(CH - 2, 0)
            _wait_gathers(1)
            _scale(1)
            _scatter(CH - 1, 1)
            _wait_scatter(0)
            _wait_scatter(1)

    with jax.named_scope("agg_edges"):
        @pl.when(c == 0)
        def _():
            _run_core(RC0, 0)

        @pl.when(c == 1)
        def _():
            _run_core(RC1, NS * RC0)

    with jax.named_scope("agg_out"):
        plsc.subcore_barrier()

        @pl.loop(0, ARS // EB)
        def _out(k):
            row = s * ARS + k * EB
            pltpu.sync_copy(acc_sh.at[pl.ds(row, EB)], rows_v.at[0])
            pltpu.sync_copy(rows_v.at[0], out_hbm.at[c].at[pl.ds(row, EB)])


def kernel(x, edge_index, edge_type, W_rel1, W_root1, b1, W_rel2, W_root2, b2):
    src = edge_index[0]
    dst = edge_index[1]
    pad = E_PAD - E

    g = edge_type * N + src
    keyb = dst * R + edge_type
    pad_i = jnp.arange(pad, dtype=jnp.int32)
    g_p = jnp.concatenate(
        [g, jnp.zeros((pad,), jnp.int32)]).reshape(ROWS, EB)
    key_p = jnp.concatenate(
        [keyb, JUNK_B + pad_i % (NB - JUNK_B)]).reshape(ROWS, EB)
    dst_p = jnp.concatenate(
        [dst, N + pad_i % (AR - N)]).reshape(ROWS, EB)

    z_cnt = jnp.zeros((NB, 16), jnp.float32)
    z_acc = jnp.zeros((AR, D), jnp.float32)

    cnt_parts = _sc_counts(key_p, z_cnt)
    inv16 = _invert_counts(cnt_parts)

    w_all1 = jnp.concatenate([W_rel1, W_root1[None]], axis=0)
    w_all2 = jnp.concatenate([W_rel2, W_root2[None]], axis=0)

    h_all1 = _transform(x, w_all1)
    parts1 = _sc_agg(h_all1.reshape((R + 1) * N, D), inv16,
                     g_p, key_p, dst_p, z_acc)
    h1 = _combine(h_all1, parts1, b1, relu=True)

    h_all2 = _transform(h1, w_all2)
    parts2 = _sc_agg(h_all2.reshape((R + 1) * N, D), inv16,
                     g_p, key_p, dst_p, z_acc)
    return _combine(h_all2, parts2, b2, relu=False)
